# Initial kernel scaffold; baseline (speedup 1.0000x reference)
import jax
import jax.numpy as jnp
from jax.experimental import pallas as pl


def kernel(G_coeffs, A_diag, A_off, neighbors, valid_mask):
    raise NotImplementedError("write your pallas kernel here")


# speedup vs baseline: 151.2351x; 151.2351x over previous
import functools

import jax
import jax.numpy as jnp
from jax import lax
from jax.experimental import pallas as pl
from jax.experimental.pallas import tpu as pltpu
from jax.experimental.pallas import tpu_sc as plsc

_B, _N, _K = 2, 100000, 24
_EPS = 0.0001
_NP = 102400
_NT = _NP // 16
_NQ = _NT // 128
_NROW = _NP // 128


def _sc_body(gt, at_, nbr, wp, mp, out,
             u_sh, v_sh, g_row, idx_buf, gv_buf, acc_buf, w_buf, m_buf,
             x_buf, part_buf):
    c = lax.axis_index("c")
    s = lax.axis_index("s")
    node0 = s * _NT
    chunk0 = s * _NQ
    f32 = jnp.float32

    pltpu.sync_copy(wp.at[c, pl.ds(node0, _NT)], w_buf)
    pltpu.sync_copy(mp.at[c, pl.ds(node0, _NT)], m_buf)
    pltpu.sync_copy(gt.at[c, 0, pl.ds(node0, _NT)], g_row)

    def init_u(i, _):
        sl = pl.ds(i * 16, 16)
        x_buf[sl] = w_buf[sl] * g_row[sl]
        return 0
    lax.fori_loop(0, _NT // 16, init_u, 0)
    pltpu.sync_copy(x_buf, u_sh.at[pl.ds(node0, _NT)])
    plsc.subcore_barrier()

    def scat_j(j, _):
        pltpu.sync_copy(gt.at[c, 1 + j, pl.ds(node0, _NT)], g_row)
        pltpu.sync_copy(nbr.at[c, j, pl.ds(chunk0, _NQ)], idx_buf)

        def q_body(q, _):
            for l in range(8):
                sl = pl.ds(q * 128 + l * 16, 16)
                gv_buf[q, pl.ds(l * 16, 16)] = w_buf[sl] * g_row[sl]
            pltpu.sync_copy(gv_buf.at[q], u_sh.at[idx_buf.at[q]], add=True)
            return 0
        lax.fori_loop(0, _NQ, q_body, 0)
        return 0
    lax.fori_loop(0, _K, scat_j, 0)
    plsc.subcore_barrier()

    pltpu.sync_copy(u_sh.at[pl.ds(node0, _NT)], x_buf)
    pltpu.sync_copy(gt.at[c, 0, pl.ds(node0, _NT)], g_row)

    def init_v(i, _):
        sl = pl.ds(i * 16, 16)
        acc_buf[sl] = x_buf[sl] * g_row[sl] + _EPS * w_buf[sl]
        return 0
    lax.fori_loop(0, _NT // 16, init_v, 0)

    def gath_j(j, _):
        pltpu.sync_copy(gt.at[c, 1 + j, pl.ds(node0, _NT)], g_row)
        pltpu.sync_copy(nbr.at[c, j, pl.ds(chunk0, _NQ)], idx_buf)

        def q_body(q, _):
            pltpu.sync_copy(u_sh.at[idx_buf.at[q]], gv_buf.at[q])
            for l in range(8):
                sl = pl.ds(q * 128 + l * 16, 16)
                acc_buf[sl] = acc_buf[sl] + gv_buf[q, pl.ds(l * 16, 16)] * g_row[sl]
            return 0
        lax.fori_loop(0, _NQ, q_body, 0)
        return 0
    lax.fori_loop(0, _K, gath_j, 0)
    pltpu.sync_copy(acc_buf, v_sh.at[pl.ds(node0, _NT)])
    plsc.subcore_barrier()

    pltpu.sync_copy(v_sh.at[pl.ds(node0, _NT)], x_buf)
    pltpu.sync_copy(at_.at[c, 0, pl.ds(node0, _NT)], g_row)

    def init_y(i, sabs):
        sl = pl.ds(i * 16, 16)
        g = g_row[sl]
        acc_buf[sl] = x_buf[sl] * g
        return sabs + jnp.abs(g)
    s_abs = lax.fori_loop(0, _NT // 16, init_y, jnp.zeros((16,), f32))

    def gath_ja(j, sabs):
        pltpu.sync_copy(at_.at[c, 1 + j, pl.ds(node0, _NT)], g_row)
        pltpu.sync_copy(nbr.at[c, j, pl.ds(chunk0, _NQ)], idx_buf)

        def q_body(q, sa):
            pltpu.sync_copy(v_sh.at[idx_buf.at[q]], gv_buf.at[q])
            for l in range(8):
                sl = pl.ds(q * 128 + l * 16, 16)
                g = g_row[sl]
                acc_buf[sl] = acc_buf[sl] + gv_buf[q, pl.ds(l * 16, 16)] * g
                sa = sa + jnp.abs(g)
            return sa
        return lax.fori_loop(0, _NQ, q_body, sabs)
    s_abs = lax.fori_loop(0, _K, gath_ja, s_abs)

    def red(i, carry):
        sy2, syw, sw2, sm = carry
        sl = pl.ds(i * 16, 16)
        m = m_buf[sl]
        my = m * acc_buf[sl]
        mw = m * w_buf[sl]
        return (sy2 + my * my, syw + my * mw, sw2 + mw * mw, sm + m)
    z16 = jnp.zeros((16,), f32)
    sy2, syw, sw2, sm = lax.fori_loop(0, _NT // 16, red, (z16, z16, z16, z16))

    part_buf[0, :] = sy2
    part_buf[1, :] = syw
    part_buf[2, :] = sw2
    part_buf[3, :] = sm
    part_buf[4, :] = s_abs
    pltpu.sync_copy(part_buf, out.at[c, s])


_mesh = plsc.VectorSubcoreMesh(core_axis_name="c", subcore_axis_name="s")

_sc_call = functools.partial(
    pl.kernel,
    out_type=jax.ShapeDtypeStruct((_B, 16, 5, 16), jnp.float32),
    mesh=_mesh,
    scratch_types=[
        pltpu.VMEM_SHARED((_NP,), jnp.float32),
        pltpu.VMEM_SHARED((_NP,), jnp.float32),
        pltpu.VMEM((_NT,), jnp.float32),
        pltpu.VMEM((_NQ, 128), jnp.int32),
        pltpu.VMEM((_NQ, 128), jnp.float32),
        pltpu.VMEM((_NT,), jnp.float32),
        pltpu.VMEM((_NT,), jnp.float32),
        pltpu.VMEM((_NT,), jnp.float32),
        pltpu.VMEM((_NT,), jnp.float32),
        pltpu.VMEM((5, 16), jnp.float32),
    ],
    compiler_params=pltpu.CompilerParams(use_tc_tiling_on_sc=False),
)(_sc_body)


def kernel(G_coeffs, A_diag, A_off, neighbors, valid_mask):
    b, n, _ = G_coeffs.shape
    pad = _NP - n
    wn = jax.random.normal(jax.random.key(42), (b, n), dtype=jnp.float32)
    m = valid_mask[:, :, 0]
    w = wn * m
    gt = jnp.pad(G_coeffs.transpose(0, 2, 1), ((0, 0), (0, 0), (0, pad)))
    at_ = jnp.pad(
        jnp.concatenate([A_diag, A_off], axis=2).transpose(0, 2, 1),
        ((0, 0), (0, 0), (0, pad)))
    nb = jnp.pad(neighbors.astype(jnp.int32).transpose(0, 2, 1),
                 ((0, 0), (0, 0), (0, pad))).reshape(b, _K, _NROW, 128)
    wp = jnp.pad(w, ((0, 0), (0, pad)))
    mp = jnp.pad(m, ((0, 0), (0, pad)))
    parts = _sc_call(gt, at_, nb, wp, mp)
    tot = parts.sum(axis=(0, 1, 3))
    sy2, syw, sw2, sm, sabs = tot[0], tot[1], tot[2], tot[3], tot[4]
    norm_a = sabs / (sm * 25 + 1e-6)
    d = norm_a + 1e-8
    loss = (sy2 / (d * d) - 2.0 * syw / d + sw2) / (sm + 1e-6)
    return loss


# speedup vs baseline: 209.8249x; 1.3874x over previous
import functools

import jax
import jax.numpy as jnp
from jax import lax
from jax.experimental import pallas as pl
from jax.experimental.pallas import tpu as pltpu
from jax.experimental.pallas import tpu_sc as plsc

_B, _N, _K = 2, 100000, 24
_EPS = 0.0001
_NP = 102400
_NT = _NP // 16
_NV = _NT // 16


def _sc_body(gt, at_, nbr, wp, mp, parts, upart, ufin, vfin,
             tbl, gbuf, ibuf, wbuf, obuf, pbuf):
    c = lax.axis_index("c")
    s = lax.axis_index("s")
    node0 = s * _NT
    f32 = jnp.float32
    z16 = jnp.zeros((16,), f32)

    pltpu.sync_copy(wp.at[c, pl.ds(node0, _NT)], wbuf)

    def zero_tbl(i, _):
        for l in range(4):
            tbl[pl.ds((i * 4 + l) * 16, 16)] = z16
        return 0
    lax.fori_loop(0, _NP // 64, zero_tbl, 0)

    pltpu.sync_copy(gt.at[c, 0, pl.ds(node0, _NT)], gbuf)

    def diag_u(i, _):
        sl = pl.ds(i * 16, 16)
        tbl[pl.ds(node0 + i * 16, 16)] = wbuf[sl] * gbuf[sl]
        return 0
    lax.fori_loop(0, _NV, diag_u, 0)

    def scat_j(j, _):
        pltpu.sync_copy(gt.at[c, 1 + j, pl.ds(node0, _NT)], gbuf)
        pltpu.sync_copy(nbr.at[c, j, pl.ds(node0, _NT)], ibuf)

        def q_body(i, _):
            for l in range(4):
                sl = pl.ds((i * 4 + l) * 16, 16)
                plsc.addupdate_scatter(tbl, [ibuf[sl]], wbuf[sl] * gbuf[sl])
            return 0
        lax.fori_loop(0, _NV // 4, q_body, 0)
        return 0
    lax.fori_loop(0, _K, scat_j, 0)

    pltpu.sync_copy(tbl, upart.at[c, s])
    plsc.subcore_barrier()

    def red_k(i, _):
        sl = pl.ds(i * 16, 16)
        acc = gbuf[sl]
        obuf[sl] = acc
        return 0
    pltpu.sync_copy(upart.at[c, 0, pl.ds(node0, _NT)], gbuf)
    lax.fori_loop(0, _NV, red_k, 0)
    for k in range(1, 16):
        pltpu.sync_copy(upart.at[c, k, pl.ds(node0, _NT)], gbuf)

        def red_k2(i, _):
            sl = pl.ds(i * 16, 16)
            obuf[sl] = obuf[sl] + gbuf[sl]
            return 0
        lax.fori_loop(0, _NV, red_k2, 0)
    pltpu.sync_copy(obuf, ufin.at[c, pl.ds(node0, _NT)])
    plsc.subcore_barrier()

    pltpu.sync_copy(ufin.at[c], tbl)
    pltpu.sync_copy(gt.at[c, 0, pl.ds(node0, _NT)], gbuf)

    def init_v(i, _):
        sl = pl.ds(i * 16, 16)
        obuf[sl] = tbl[pl.ds(node0 + i * 16, 16)] * gbuf[sl] + _EPS * wbuf[sl]
        return 0
    lax.fori_loop(0, _NV, init_v, 0)

    def gath_j(j, _):
        pltpu.sync_copy(gt.at[c, 1 + j, pl.ds(node0, _NT)], gbuf)
        pltpu.sync_copy(nbr.at[c, j, pl.ds(node0, _NT)], ibuf)

        def q_body(i, _):
            for l in range(4):
                sl = pl.ds((i * 4 + l) * 16, 16)
                obuf[sl] = obuf[sl] + plsc.load_gather(tbl, [ibuf[sl]]) * gbuf[sl]
            return 0
        lax.fori_loop(0, _NV // 4, q_body, 0)
        return 0
    lax.fori_loop(0, _K, gath_j, 0)
    pltpu.sync_copy(obuf, vfin.at[c, pl.ds(node0, _NT)])
    plsc.subcore_barrier()

    pltpu.sync_copy(vfin.at[c], tbl)
    pltpu.sync_copy(at_.at[c, 0, pl.ds(node0, _NT)], gbuf)

    def init_y(i, sabs):
        sl = pl.ds(i * 16, 16)
        g = gbuf[sl]
        obuf[sl] = tbl[pl.ds(node0 + i * 16, 16)] * g
        return sabs + jnp.abs(g)
    s_abs = lax.fori_loop(0, _NV, init_y, z16)

    def gath_ja(j, sabs):
        pltpu.sync_copy(at_.at[c, 1 + j, pl.ds(node0, _NT)], gbuf)
        pltpu.sync_copy(nbr.at[c, j, pl.ds(node0, _NT)], ibuf)

        def q_body(i, sa):
            for l in range(4):
                sl = pl.ds((i * 4 + l) * 16, 16)
                g = gbuf[sl]
                obuf[sl] = obuf[sl] + plsc.load_gather(tbl, [ibuf[sl]]) * g
                sa = sa + jnp.abs(g)
            return sa
        return lax.fori_loop(0, _NV // 4, q_body, sabs)
    s_abs = lax.fori_loop(0, _K, gath_ja, s_abs)

    pltpu.sync_copy(mp.at[c, pl.ds(node0, _NT)], gbuf)

    def red(i, carry):
        sy2, syw, sw2, sm = carry
        sl = pl.ds(i * 16, 16)
        m = gbuf[sl]
        my = m * obuf[sl]
        mw = m * wbuf[sl]
        return (sy2 + my * my, syw + my * mw, sw2 + mw * mw, sm + m)
    sy2, syw, sw2, sm = lax.fori_loop(0, _NV, red, (z16, z16, z16, z16))

    pbuf[0, :] = sy2
    pbuf[1, :] = syw
    pbuf[2, :] = sw2
    pbuf[3, :] = sm
    pbuf[4, :] = s_abs
    pltpu.sync_copy(pbuf, parts.at[c, s])


_mesh = plsc.VectorSubcoreMesh(core_axis_name="c", subcore_axis_name="s")

_sc_call = functools.partial(
    pl.kernel,
    out_type=(
        jax.ShapeDtypeStruct((_B, 16, 5, 16), jnp.float32),
        jax.ShapeDtypeStruct((_B, 16, _NP), jnp.float32),
        jax.ShapeDtypeStruct((_B, _NP), jnp.float32),
        jax.ShapeDtypeStruct((_B, _NP), jnp.float32),
    ),
    mesh=_mesh,
    scratch_types=[
        pltpu.VMEM((_NP,), jnp.float32),
        pltpu.VMEM((_NT,), jnp.float32),
        pltpu.VMEM((_NT,), jnp.int32),
        pltpu.VMEM((_NT,), jnp.float32),
        pltpu.VMEM((_NT,), jnp.float32),
        pltpu.VMEM((5, 16), jnp.float32),
    ],
    compiler_params=pltpu.CompilerParams(use_tc_tiling_on_sc=False,
                                         needs_layout_passes=False),
)(_sc_body)


def kernel(G_coeffs, A_diag, A_off, neighbors, valid_mask):
    b, n, _ = G_coeffs.shape
    pad = _NP - n
    wn = jax.random.normal(jax.random.key(42), (b, n), dtype=jnp.float32)
    m = valid_mask[:, :, 0]
    w = wn * m
    gt = jnp.pad(G_coeffs.transpose(0, 2, 1), ((0, 0), (0, 0), (0, pad)))
    at_ = jnp.pad(
        jnp.concatenate([A_diag, A_off], axis=2).transpose(0, 2, 1),
        ((0, 0), (0, 0), (0, pad)))
    nb = jnp.pad(neighbors.astype(jnp.int32).transpose(0, 2, 1),
                 ((0, 0), (0, 0), (0, pad)))
    wp = jnp.pad(w, ((0, 0), (0, pad)))
    mp = jnp.pad(m, ((0, 0), (0, pad)))
    parts, _, _, _ = _sc_call(gt, at_, nb, wp, mp)
    tot = parts.sum(axis=(0, 1, 3))
    sy2, syw, sw2, sm, sabs = tot[0], tot[1], tot[2], tot[3], tot[4]
    norm_a = sabs / (sm * 25 + 1e-6)
    d = norm_a + 1e-8
    loss = (sy2 / (d * d) - 2.0 * syw / d + sw2) / (sm + 1e-6)
    return loss


# speedup vs baseline: 272.9294x; 1.3007x over previous
import functools

import jax
import jax.numpy as jnp
from jax import lax
from jax.experimental import pallas as pl
from jax.experimental.pallas import tpu as pltpu
from jax.experimental.pallas import tpu_sc as plsc

_B, _N, _K = 2, 100000, 24
_EPS = 0.0001
_NP = 102400
_NT = _NP // 16
_NS = _NT // 2
_NG = _NS // 16


def _sc_body(gt, at_, nbr, wp, mp, parts, upart, ufin, vfin,
             tbl, gbufs, ibufs, wbuf, obuf, pbuf,
             gsem0, gsem1, isem0, isem1):
    c = lax.axis_index("c")
    s = lax.axis_index("s")
    node0 = s * _NT
    f32 = jnp.float32
    z16 = jnp.zeros((16,), f32)
    gsems = (gsem0, gsem1)
    isems = (isem0, isem1)

    def start_slab(coeff, row, h, b):
        off = node0 + h * _NS
        pltpu.async_copy(coeff.at[c, row, pl.ds(off, _NS)], gbufs.at[b],
                         gsems[b])
        pltpu.async_copy(nbr.at[c, row - 1, pl.ds(off, _NS)], ibufs.at[b],
                         isems[b])

    def wait_slab(coeff, row, h, b):
        off = node0 + h * _NS
        pltpu.make_async_copy(coeff.at[c, row, pl.ds(off, _NS)], gbufs.at[b],
                              gsems[b]).wait()
        pltpu.make_async_copy(nbr.at[c, row - 1, pl.ds(off, _NS)],
                              ibufs.at[b], isems[b]).wait()

    def pipeline(coeff, compute, carry_init):
        start_slab(coeff, 1, 0, 0)

        def body(j, carry):
            start_slab(coeff, j + 1, 1, 1)
            wait_slab(coeff, j + 1, 0, 0)
            carry = compute(0, 0, carry)

            @pl.when(j + 1 < _K)
            def _():
                start_slab(coeff, j + 2, 0, 0)
            wait_slab(coeff, j + 1, 1, 1)
            carry = compute(1, 1, carry)
            return carry
        return lax.fori_loop(0, _K, body, carry_init)

    pltpu.sync_copy(wp.at[c, pl.ds(node0, _NT)], wbuf)

    def zero_tbl(i, _):
        for l in range(8):
            tbl[pl.ds((i * 8 + l) * 16, 16)] = z16
        return 0
    lax.fori_loop(0, _NP // 128, zero_tbl, 0)

    pltpu.sync_copy(gt.at[c, 0, pl.ds(node0, _NS)], gbufs.at[0])
    pltpu.sync_copy(gt.at[c, 0, pl.ds(node0 + _NS, _NS)], gbufs.at[1])
    for h in range(2):
        def diag_u(i, _, h=h):
            for l in range(8):
                g = i * 8 + l
                sl = pl.ds(g * 16, 16)
                wsl = pl.ds(h * _NS + g * 16, 16)
                tbl[pl.ds(node0 + h * _NS + g * 16, 16)] = \
                    wbuf[wsl] * gbufs[h, sl]
            return 0
        lax.fori_loop(0, _NG // 8, diag_u, 0)

    def scat(h, b, carry):
        woff = h * _NS

        def q_body(i, _):
            for l in range(8):
                g = i * 8 + l
                sl = pl.ds(g * 16, 16)
                wsl = pl.ds(woff + g * 16, 16)
                plsc.addupdate_scatter(tbl, [ibufs[b, sl]],
                                       wbuf[wsl] * gbufs[b, sl])
            return 0
        lax.fori_loop(0, _NG // 8, q_body, 0)
        return carry
    pipeline(gt, scat, 0)

    pltpu.sync_copy(tbl, upart.at[c, s])
    plsc.subcore_barrier()

    def start_part(k, h, b):
        pltpu.async_copy(upart.at[c, k, pl.ds(node0 + h * _NS, _NS)],
                         gbufs.at[b], gsems[b])

    def wait_part(k, h, b):
        pltpu.make_async_copy(upart.at[c, k, pl.ds(node0 + h * _NS, _NS)],
                              gbufs.at[b], gsems[b]).wait()

    start_part(0, 0, 0)

    def red_body(k, _):
        start_part(k, 1, 1)
        wait_part(k, 0, 0)

        def add0(i, _):
            for l in range(8):
                g = i * 8 + l
                sl = pl.ds(g * 16, 16)
                obuf[sl] = jnp.where(k == 0, z16, obuf[sl]) + gbufs[0, sl]
            return 0
        lax.fori_loop(0, _NG // 8, add0, 0)

        @pl.when(k + 1 < 16)
        def _():
            start_part(k + 1, 0, 0)
        wait_part(k, 1, 1)

        def add1(i, _):
            for l in range(8):
                g = i * 8 + l
                sl = pl.ds(g * 16, 16)
                osl = pl.ds(_NS + g * 16, 16)
                obuf[osl] = jnp.where(k == 0, z16, obuf[osl]) + gbufs[1, sl]
            return 0
        lax.fori_loop(0, _NG // 8, add1, 0)
        return 0
    lax.fori_loop(0, 16, red_body, 0)
    pltpu.sync_copy(obuf, ufin.at[c, pl.ds(node0, _NT)])
    plsc.subcore_barrier()

    pltpu.sync_copy(ufin.at[c], tbl)
    pltpu.sync_copy(gt.at[c, 0, pl.ds(node0, _NS)], gbufs.at[0])
    pltpu.sync_copy(gt.at[c, 0, pl.ds(node0 + _NS, _NS)], gbufs.at[1])
    for h in range(2):
        def init_v(i, _, h=h):
            for l in range(8):
                g = i * 8 + l
                sl = pl.ds(g * 16, 16)
                osl = pl.ds(h * _NS + g * 16, 16)
                obuf[osl] = (tbl[pl.ds(node0 + h * _NS + g * 16, 16)]
                             * gbufs[h, sl] + _EPS * wbuf[osl])
            return 0
        lax.fori_loop(0, _NG // 8, init_v, 0)

    def gath(h, b, carry):
        ooff = h * _NS

        def q_body(i, _):
            for l in range(8):
                g = i * 8 + l
                sl = pl.ds(g * 16, 16)
                osl = pl.ds(ooff + g * 16, 16)
                obuf[osl] = obuf[osl] + plsc.load_gather(
                    tbl, [ibufs[b, sl]]) * gbufs[b, sl]
            return 0
        lax.fori_loop(0, _NG // 8, q_body, 0)
        return carry
    pipeline(gt, gath, 0)
    pltpu.sync_copy(obuf, vfin.at[c, pl.ds(node0, _NT)])
    plsc.subcore_barrier()

    pltpu.sync_copy(vfin.at[c], tbl)
    pltpu.sync_copy(at_.at[c, 0, pl.ds(node0, _NS)], gbufs.at[0])
    pltpu.sync_copy(at_.at[c, 0, pl.ds(node0 + _NS, _NS)], gbufs.at[1])
    s_abs = z16
    for h in range(2):
        def init_y(i, sabs, h=h):
            for l in range(8):
                g = i * 8 + l
                sl = pl.ds(g * 16, 16)
                osl = pl.ds(h * _NS + g * 16, 16)
                gv = gbufs[h, sl]
                obuf[osl] = tbl[pl.ds(node0 + h * _NS + g * 16, 16)] * gv
                sabs = sabs + jnp.abs(gv)
            return sabs
        s_abs = lax.fori_loop(0, _NG // 8, init_y, s_abs)

    def gath_a(h, b, sabs):
        ooff = h * _NS

        def q_body(i, sa):
            for l in range(8):
                g = i * 8 + l
                sl = pl.ds(g * 16, 16)
                osl = pl.ds(ooff + g * 16, 16)
                gv = gbufs[b, sl]
                obuf[osl] = obuf[osl] + plsc.load_gather(
                    tbl, [ibufs[b, sl]]) * gv
                sa = sa + jnp.abs(gv)
            return sa
        return lax.fori_loop(0, _NG // 8, q_body, sabs)
    s_abs = pipeline(at_, gath_a, s_abs)

    pltpu.sync_copy(mp.at[c, pl.ds(node0, _NS)], gbufs.at[0])
    pltpu.sync_copy(mp.at[c, pl.ds(node0 + _NS, _NS)], gbufs.at[1])

    def red0(i, carry):
        sy2, syw, sw2, sm = carry
        for l in range(8):
            g = i * 8 + l
            sl = pl.ds(g * 16, 16)
            m = gbufs[0, sl]
            my = m * obuf[sl]
            mw = m * wbuf[sl]
            sy2 = sy2 + my * my
            syw = syw + my * mw
            sw2 = sw2 + mw * mw
            sm = sm + m
        return (sy2, syw, sw2, sm)
    acc = lax.fori_loop(0, _NG // 8, red0, (z16, z16, z16, z16))

    def red1(i, carry):
        sy2, syw, sw2, sm = carry
        for l in range(8):
            g = i * 8 + l
            sl = pl.ds(g * 16, 16)
            osl = pl.ds(_NS + g * 16, 16)
            m = gbufs[1, sl]
            my = m * obuf[osl]
            mw = m * wbuf[osl]
            sy2 = sy2 + my * my
            syw = syw + my * mw
            sw2 = sw2 + mw * mw
            sm = sm + m
        return (sy2, syw, sw2, sm)
    sy2, syw, sw2, sm = lax.fori_loop(0, _NG // 8, red1, acc)

    pbuf[0, :] = sy2
    pbuf[1, :] = syw
    pbuf[2, :] = sw2
    pbuf[3, :] = sm
    pbuf[4, :] = s_abs
    pltpu.sync_copy(pbuf, parts.at[c, s])


_mesh = plsc.VectorSubcoreMesh(core_axis_name="c", subcore_axis_name="s")

_sc_call = functools.partial(
    pl.kernel,
    out_type=(
        jax.ShapeDtypeStruct((_B, 16, 5, 16), jnp.float32),
        jax.ShapeDtypeStruct((_B, 16, _NP), jnp.float32),
        jax.ShapeDtypeStruct((_B, _NP), jnp.float32),
        jax.ShapeDtypeStruct((_B, _NP), jnp.float32),
    ),
    mesh=_mesh,
    scratch_types=[
        pltpu.VMEM((_NP,), jnp.float32),
        pltpu.VMEM((2, _NS), jnp.float32),
        pltpu.VMEM((2, _NS), jnp.int32),
        pltpu.VMEM((_NT,), jnp.float32),
        pltpu.VMEM((_NT,), jnp.float32),
        pltpu.VMEM((5, 16), jnp.float32),
        pltpu.SemaphoreType.DMA,
        pltpu.SemaphoreType.DMA,
        pltpu.SemaphoreType.DMA,
        pltpu.SemaphoreType.DMA,
    ],
    compiler_params=pltpu.CompilerParams(use_tc_tiling_on_sc=False,
                                         needs_layout_passes=False),
)(_sc_body)


def kernel(G_coeffs, A_diag, A_off, neighbors, valid_mask):
    b, n, _ = G_coeffs.shape
    pad = _NP - n
    wn = jax.random.normal(jax.random.key(42), (b, n), dtype=jnp.float32)
    m = valid_mask[:, :, 0]
    w = wn * m
    gt = jnp.pad(G_coeffs.transpose(0, 2, 1), ((0, 0), (0, 0), (0, pad)))
    at_ = jnp.pad(
        jnp.concatenate([A_diag, A_off], axis=2).transpose(0, 2, 1),
        ((0, 0), (0, 0), (0, pad)))
    nb = jnp.pad(neighbors.astype(jnp.int32).transpose(0, 2, 1),
                 ((0, 0), (0, 0), (0, pad)))
    wp = jnp.pad(w, ((0, 0), (0, pad)))
    mp = jnp.pad(m, ((0, 0), (0, pad)))
    parts, _, _, _ = _sc_call(gt, at_, nb, wp, mp)
    tot = parts.sum(axis=(0, 1, 3))
    sy2, syw, sw2, sm, sabs = tot[0], tot[1], tot[2], tot[3], tot[4]
    norm_a = sabs / (sm * 25 + 1e-6)
    d = norm_a + 1e-8
    loss = (sy2 / (d * d) - 2.0 * syw / d + sw2) / (sm + 1e-6)
    return loss


# speedup vs baseline: 274.5544x; 1.0060x over previous
import functools

import jax
import jax.numpy as jnp
from jax import lax
from jax.experimental import pallas as pl
from jax.experimental.pallas import tpu as pltpu
from jax.experimental.pallas import tpu_sc as plsc

_B, _N, _K = 2, 100000, 24
_EPS = 0.0001
_NP = 102400
_NT = _NP // 16
_NB = _NT // 128
_NBLK = _NP // 128
_GP = 25 * 128
_IP = 24 * 128
_RQ = _NT // 4


def _sc_body(gq, aq, nbq, wp, mp, parts, upart, ufin, vfin,
             tbl, gbufs, ibufs, wbuf, obuf, pbuf,
             gsem0, gsem1, isem0, isem1):
    c = lax.axis_index("c")
    s = lax.axis_index("s")
    node0 = s * _NT
    blk0 = s * _NB
    f32 = jnp.float32
    z16 = jnp.zeros((16,), f32)
    gsems = (gsem0, gsem1)
    isems = (isem0, isem1)

    def start_blk(coeff, bi, b):
        pltpu.async_copy(coeff.at[c, blk0 + bi], gbufs.at[b], gsems[b])
        pltpu.async_copy(nbq.at[c, blk0 + bi], ibufs.at[b], isems[b])

    def wait_blk(coeff, bi, b):
        pltpu.make_async_copy(coeff.at[c, blk0 + bi], gbufs.at[b],
                              gsems[b]).wait()
        pltpu.make_async_copy(nbq.at[c, blk0 + bi], ibufs.at[b],
                              isems[b]).wait()

    def pipeline(coeff, process, carry_init):
        start_blk(coeff, 0, 0)

        def body(t, carry):
            bi0 = 2 * t
            start_blk(coeff, bi0 + 1, 1)
            wait_blk(coeff, bi0, 0)
            carry = process(bi0, 0, carry)

            @pl.when(bi0 + 2 < _NB)
            def _():
                start_blk(coeff, bi0 + 2, 0)
            wait_blk(coeff, bi0 + 1, 1)
            carry = process(bi0 + 1, 1, carry)
            return carry
        return lax.fori_loop(0, _NB // 2, body, carry_init)

    pltpu.sync_copy(wp.at[c, pl.ds(node0, _NT)], wbuf)

    def zero_tbl(i, _):
        for l in range(8):
            tbl[pl.ds((i * 8 + l) * 16, 16)] = z16
        return 0
    lax.fori_loop(0, _NP // 128, zero_tbl, 0)

    def scat(bi, b, carry):
        boff = bi * 128
        for l in range(8):
            sl = pl.ds(l * 16, 16)
            wsl = pl.ds(boff + l * 16, 16)
            tbl[pl.ds(node0 + boff + l * 16, 16)] = wbuf[wsl] * gbufs[b, sl]

        def row(j, _):
            ro = j * 128
            io = ro - 128
            for l in range(8):
                idx = ibufs[b, pl.ds(io + l * 16, 16)]
                val = (wbuf[pl.ds(boff + l * 16, 16)]
                       * gbufs[b, pl.ds(ro + l * 16, 16)])
                plsc.addupdate_scatter(tbl, [idx], val)
            return 0
        lax.fori_loop(1, 25, row, 0)
        return carry
    pipeline(gq, scat, 0)

    pltpu.sync_copy(tbl, upart.at[c, s])
    plsc.subcore_barrier()

    pltpu.sync_copy(upart.at[c, 0, pl.ds(node0, _NT)], obuf)

    def start_part(k, h, b):
        pltpu.async_copy(upart.at[c, k, pl.ds(node0 + h * _RQ, _RQ)],
                         gbufs.at[b, pl.ds(0, _RQ)], gsems[b])

    def wait_part(k, h, b):
        pltpu.make_async_copy(upart.at[c, k, pl.ds(node0 + h * _RQ, _RQ)],
                              gbufs.at[b, pl.ds(0, _RQ)], gsems[b]).wait()

    def red_pair(t, _):
        q0 = 2 * t
        q1 = q0 + 1
        q2 = q0 + 2
        q3 = q0 + 3

        wait_part(q0 // 4 + 1, q0 % 4, 0)

        def add0(i, _):
            for l in range(8):
                g = i * 8 + l
                osl = pl.ds((q0 % 4) * _RQ + g * 16, 16)
                obuf[osl] = obuf[osl] + gbufs[0, pl.ds(g * 16, 16)]
            return 0
        lax.fori_loop(0, _RQ // 128, add0, 0)

        @pl.when(q2 < 60)
        def _():
            start_part(q2 // 4 + 1, q2 % 4, 0)

        wait_part(q1 // 4 + 1, q1 % 4, 1)

        def add1(i, _):
            for l in range(8):
                g = i * 8 + l
                osl = pl.ds((q1 % 4) * _RQ + g * 16, 16)
                obuf[osl] = obuf[osl] + gbufs[1, pl.ds(g * 16, 16)]
            return 0
        lax.fori_loop(0, _RQ // 128, add1, 0)

        @pl.when(q3 < 60)
        def _():
            start_part(q3 // 4 + 1, q3 % 4, 1)
        return 0

    start_part(1, 0, 0)
    start_part(1, 1, 1)
    lax.fori_loop(0, 30, red_pair, 0)
    pltpu.sync_copy(obuf, ufin.at[c, pl.ds(node0, _NT)])
    plsc.subcore_barrier()

    pltpu.sync_copy(ufin.at[c], tbl)

    def gath(bi, b, carry):
        boff = bi * 128
        acc = []
        for l in range(8):
            sl = pl.ds(l * 16, 16)
            wsl = pl.ds(boff + l * 16, 16)
            acc.append(tbl[pl.ds(node0 + boff + l * 16, 16)] * gbufs[b, sl]
                       + _EPS * wbuf[wsl])

        def row(j, acc):
            ro = j * 128
            io = ro - 128
            out = []
            for l in range(8):
                idx = ibufs[b, pl.ds(io + l * 16, 16)]
                g = gbufs[b, pl.ds(ro + l * 16, 16)]
                out.append(acc[l] + plsc.load_gather(tbl, [idx]) * g)
            return tuple(out)
        acc = lax.fori_loop(1, 25, row, tuple(acc))
        for l in range(8):
            obuf[pl.ds(boff + l * 16, 16)] = acc[l]
        return carry
    pipeline(gq, gath, 0)
    pltpu.sync_copy(obuf, vfin.at[c, pl.ds(node0, _NT)])
    plsc.subcore_barrier()

    pltpu.sync_copy(vfin.at[c], tbl)

    def gath_a(bi, b, sabs):
        boff = bi * 128
        acc = []
        for l in range(8):
            sl = pl.ds(l * 16, 16)
            g = gbufs[b, sl]
            acc.append(tbl[pl.ds(node0 + boff + l * 16, 16)] * g)
            sabs = sabs + jnp.abs(g)

        def row(j, carry):
            acc, sa = carry
            ro = j * 128
            io = ro - 128
            out = []
            for l in range(8):
                idx = ibufs[b, pl.ds(io + l * 16, 16)]
                g = gbufs[b, pl.ds(ro + l * 16, 16)]
                out.append(acc[l] + plsc.load_gather(tbl, [idx]) * g)
                sa = sa + jnp.abs(g)
            return (tuple(out), sa)
        acc, sabs = lax.fori_loop(1, 25, row, (tuple(acc), sabs))
        for l in range(8):
            obuf[pl.ds(boff + l * 16, 16)] = acc[l]
        return sabs
    s_abs = pipeline(aq, gath_a, z16)

    pltpu.sync_copy(mp.at[c, pl.ds(node0, _NT // 2)],
                    gbufs.at[0, pl.ds(0, _NT // 2)])
    pltpu.sync_copy(mp.at[c, pl.ds(node0 + _NT // 2, _NT // 2)],
                    gbufs.at[1, pl.ds(0, _NT // 2)])

    def red0(i, carry):
        sy2, syw, sw2, sm = carry
        for l in range(8):
            g = i * 8 + l
            sl = pl.ds(g * 16, 16)
            m = gbufs[0, sl]
            my = m * obuf[sl]
            mw = m * wbuf[sl]
            sy2 = sy2 + my * my
            syw = syw + my * mw
            sw2 = sw2 + mw * mw
            sm = sm + m
        return (sy2, syw, sw2, sm)
    accr = lax.fori_loop(0, _NT // 256, red0, (z16, z16, z16, z16))

    def red1(i, carry):
        sy2, syw, sw2, sm = carry
        for l in range(8):
            g = i * 8 + l
            sl = pl.ds(g * 16, 16)
            osl = pl.ds(_NT // 2 + g * 16, 16)
            m = gbufs[1, sl]
            my = m * obuf[osl]
            mw = m * wbuf[osl]
            sy2 = sy2 + my * my
            syw = syw + my * mw
            sw2 = sw2 + mw * mw
            sm = sm + m
        return (sy2, syw, sw2, sm)
    sy2, syw, sw2, sm = lax.fori_loop(0, _NT // 256, red1, accr)

    pbuf[0, :] = sy2
    pbuf[1, :] = syw
    pbuf[2, :] = sw2
    pbuf[3, :] = sm
    pbuf[4, :] = s_abs
    pltpu.sync_copy(pbuf, parts.at[c, s])


_mesh = plsc.VectorSubcoreMesh(core_axis_name="c", subcore_axis_name="s")

_sc_call = functools.partial(
    pl.kernel,
    out_type=(
        jax.ShapeDtypeStruct((_B, 16, 5, 16), jnp.float32),
        jax.ShapeDtypeStruct((_B, 16, _NP), jnp.float32),
        jax.ShapeDtypeStruct((_B, _NP), jnp.float32),
        jax.ShapeDtypeStruct((_B, _NP), jnp.float32),
    ),
    mesh=_mesh,
    scratch_types=[
        pltpu.VMEM((_NP,), jnp.float32),
        pltpu.VMEM((2, _GP), jnp.float32),
        pltpu.VMEM((2, _IP), jnp.int32),
        pltpu.VMEM((_NT,), jnp.float32),
        pltpu.VMEM((_NT,), jnp.float32),
        pltpu.VMEM((5, 16), jnp.float32),
        pltpu.SemaphoreType.DMA,
        pltpu.SemaphoreType.DMA,
        pltpu.SemaphoreType.DMA,
        pltpu.SemaphoreType.DMA,
    ],
    compiler_params=pltpu.CompilerParams(use_tc_tiling_on_sc=False,
                                         needs_layout_passes=False),
)(_sc_body)


def kernel(G_coeffs, A_diag, A_off, neighbors, valid_mask):
    b, n, _ = G_coeffs.shape
    pad = _NP - n
    wn = jax.random.normal(jax.random.key(42), (b, n), dtype=jnp.float32)
    m = valid_mask[:, :, 0]
    w = wn * m

    def panels(x, width):
        xp = jnp.pad(x, ((0, 0), (0, pad), (0, 0)))
        return xp.reshape(b, _NBLK, 128, width).swapaxes(2, 3).reshape(
            b, _NBLK, width * 128)

    gq = panels(G_coeffs, 25)
    aq = panels(jnp.concatenate([A_diag, A_off], axis=2), 25)
    nbq = panels(neighbors.astype(jnp.int32), 24)
    wp = jnp.pad(w, ((0, 0), (0, pad)))
    mp = jnp.pad(m, ((0, 0), (0, pad)))
    parts, _, _, _ = _sc_call(gq, aq, nbq, wp, mp)
    tot = parts.sum(axis=(0, 1, 3))
    sy2, syw, sw2, sm, sabs = tot[0], tot[1], tot[2], tot[3], tot[4]
    norm_a = sabs / (sm * 25 + 1e-6)
    d = norm_a + 1e-8
    loss = (sy2 / (d * d) - 2.0 * syw / d + sw2) / (sm + 1e-6)
    return loss


# speedup vs baseline: 274.7470x; 1.0007x over previous
import functools

import jax
import jax.numpy as jnp
from jax import lax
from jax.experimental import pallas as pl
from jax.experimental.pallas import tpu as pltpu
from jax.experimental.pallas import tpu_sc as plsc

_B, _N, _K = 2, 100000, 24
_EPS = 0.0001
_NP = 102400
_NT = _NP // 16
_NB = _NT // 128
_NBLK = _NP // 128
_GP = 25 * 128
_IP = 24 * 128
_RQ = _NT // 4


def _sc_body(gq, aq, nbq, wp, mp, parts, upart, ufin, vfin,
             tbl, gbufs, ibufs, wbuf, obuf, pbuf,
             gsem0, gsem1, isem0, isem1):
    c = lax.axis_index("c")
    s = lax.axis_index("s")
    node0 = s * _NT
    blk0 = s * _NB
    f32 = jnp.float32
    z16 = jnp.zeros((16,), f32)
    gsems = (gsem0, gsem1)
    isems = (isem0, isem1)

    def start_blk(coeff, bi, b):
        pltpu.async_copy(coeff.at[c, blk0 + bi], gbufs.at[b], gsems[b])
        pltpu.async_copy(nbq.at[c, blk0 + bi], ibufs.at[b], isems[b])

    def wait_blk(coeff, bi, b):
        pltpu.make_async_copy(coeff.at[c, blk0 + bi], gbufs.at[b],
                              gsems[b]).wait()
        pltpu.make_async_copy(nbq.at[c, blk0 + bi], ibufs.at[b],
                              isems[b]).wait()

    def pipeline(coeff, process, carry_init):
        start_blk(coeff, 0, 0)

        def body(t, carry):
            bi0 = 2 * t
            start_blk(coeff, bi0 + 1, 1)
            wait_blk(coeff, bi0, 0)
            carry = process(bi0, 0, carry)

            @pl.when(bi0 + 2 < _NB)
            def _():
                start_blk(coeff, bi0 + 2, 0)
            wait_blk(coeff, bi0 + 1, 1)
            carry = process(bi0 + 1, 1, carry)
            return carry
        return lax.fori_loop(0, _NB // 2, body, carry_init)

    pltpu.sync_copy(wp.at[c, pl.ds(node0, _NT)], wbuf)

    def zero_tbl(i, _):
        for l in range(8):
            tbl[pl.ds((i * 8 + l) * 16, 16)] = z16
        return 0
    lax.fori_loop(0, _NP // 128, zero_tbl, 0)

    def scat(bi, b, carry):
        boff = bi * 128
        for l in range(8):
            sl = pl.ds(l * 16, 16)
            wsl = pl.ds(boff + l * 16, 16)
            plsc.addupdate(tbl.at[pl.ds(node0 + boff + l * 16, 16)],
                           wbuf[wsl] * gbufs[b, sl])

        def row(j, _):
            ro = j * 128
            io = ro - 128
            for l in range(8):
                idx = ibufs[b, pl.ds(io + l * 16, 16)]
                val = (wbuf[pl.ds(boff + l * 16, 16)]
                       * gbufs[b, pl.ds(ro + l * 16, 16)])
                plsc.addupdate_scatter(tbl, [idx], val)
            return 0
        lax.fori_loop(1, 25, row, 0)
        return carry
    pipeline(gq, scat, 0)

    pltpu.sync_copy(tbl, upart.at[c, s])
    plsc.subcore_barrier()

    pltpu.sync_copy(upart.at[c, 0, pl.ds(node0, _NT)], obuf)

    def start_part(k, h, b):
        pltpu.async_copy(upart.at[c, k, pl.ds(node0 + h * _RQ, _RQ)],
                         gbufs.at[b, pl.ds(0, _RQ)], gsems[b])

    def wait_part(k, h, b):
        pltpu.make_async_copy(upart.at[c, k, pl.ds(node0 + h * _RQ, _RQ)],
                              gbufs.at[b, pl.ds(0, _RQ)], gsems[b]).wait()

    def red_pair(t, _):
        q0 = 2 * t
        q1 = q0 + 1
        q2 = q0 + 2
        q3 = q0 + 3

        wait_part(q0 // 4 + 1, q0 % 4, 0)

        def add0(i, _):
            for l in range(8):
                g = i * 8 + l
                osl = pl.ds((q0 % 4) * _RQ + g * 16, 16)
                obuf[osl] = obuf[osl] + gbufs[0, pl.ds(g * 16, 16)]
            return 0
        lax.fori_loop(0, _RQ // 128, add0, 0)

        @pl.when(q2 < 60)
        def _():
            start_part(q2 // 4 + 1, q2 % 4, 0)

        wait_part(q1 // 4 + 1, q1 % 4, 1)

        def add1(i, _):
            for l in range(8):
                g = i * 8 + l
                osl = pl.ds((q1 % 4) * _RQ + g * 16, 16)
                obuf[osl] = obuf[osl] + gbufs[1, pl.ds(g * 16, 16)]
            return 0
        lax.fori_loop(0, _RQ // 128, add1, 0)

        @pl.when(q3 < 60)
        def _():
            start_part(q3 // 4 + 1, q3 % 4, 1)
        return 0

    start_part(1, 0, 0)
    start_part(1, 1, 1)
    lax.fori_loop(0, 30, red_pair, 0)
    pltpu.sync_copy(obuf, ufin.at[c, pl.ds(node0, _NT)])
    plsc.subcore_barrier()

    pltpu.sync_copy(ufin.at[c], tbl)

    def gath(bi, b, carry):
        boff = bi * 128
        acc = []
        for l in range(8):
            sl = pl.ds(l * 16, 16)
            wsl = pl.ds(boff + l * 16, 16)
            acc.append(tbl[pl.ds(node0 + boff + l * 16, 16)] * gbufs[b, sl]
                       + _EPS * wbuf[wsl])

        def row(j, acc):
            ro = j * 128
            io = ro - 128
            out = []
            for l in range(8):
                idx = ibufs[b, pl.ds(io + l * 16, 16)]
                g = gbufs[b, pl.ds(ro + l * 16, 16)]
                out.append(acc[l] + plsc.load_gather(tbl, [idx]) * g)
            return tuple(out)
        acc = lax.fori_loop(1, 25, row, tuple(acc))
        for l in range(8):
            obuf[pl.ds(boff + l * 16, 16)] = acc[l]
        return carry
    pipeline(gq, gath, 0)
    pltpu.sync_copy(obuf, vfin.at[c, pl.ds(node0, _NT)])
    plsc.subcore_barrier()

    pltpu.sync_copy(vfin.at[c], tbl)

    def gath_a(bi, b, sabs):
        boff = bi * 128
        acc = []
        for l in range(8):
            sl = pl.ds(l * 16, 16)
            g = gbufs[b, sl]
            acc.append(tbl[pl.ds(node0 + boff + l * 16, 16)] * g)
            sabs = sabs + jnp.abs(g)

        def row(j, carry):
            acc, sa = carry
            ro = j * 128
            io = ro - 128
            out = []
            for l in range(8):
                idx = ibufs[b, pl.ds(io + l * 16, 16)]
                g = gbufs[b, pl.ds(ro + l * 16, 16)]
                out.append(acc[l] + plsc.load_gather(tbl, [idx]) * g)
                sa = sa + jnp.abs(g)
            return (tuple(out), sa)
        acc, sabs = lax.fori_loop(1, 25, row, (tuple(acc), sabs))
        for l in range(8):
            obuf[pl.ds(boff + l * 16, 16)] = acc[l]
        return sabs
    s_abs = pipeline(aq, gath_a, z16)

    pltpu.sync_copy(mp.at[c, pl.ds(node0, _NT // 2)],
                    gbufs.at[0, pl.ds(0, _NT // 2)])
    pltpu.sync_copy(mp.at[c, pl.ds(node0 + _NT // 2, _NT // 2)],
                    gbufs.at[1, pl.ds(0, _NT // 2)])

    def red0(i, carry):
        sy2, syw, sw2, sm = carry
        for l in range(8):
            g = i * 8 + l
            sl = pl.ds(g * 16, 16)
            m = gbufs[0, sl]
            my = m * obuf[sl]
            mw = m * wbuf[sl]
            sy2 = sy2 + my * my
            syw = syw + my * mw
            sw2 = sw2 + mw * mw
            sm = sm + m
        return (sy2, syw, sw2, sm)
    accr = lax.fori_loop(0, _NT // 256, red0, (z16, z16, z16, z16))

    def red1(i, carry):
        sy2, syw, sw2, sm = carry
        for l in range(8):
            g = i * 8 + l
            sl = pl.ds(g * 16, 16)
            osl = pl.ds(_NT // 2 + g * 16, 16)
            m = gbufs[1, sl]
            my = m * obuf[osl]
            mw = m * wbuf[osl]
            sy2 = sy2 + my * my
            syw = syw + my * mw
            sw2 = sw2 + mw * mw
            sm = sm + m
        return (sy2, syw, sw2, sm)
    sy2, syw, sw2, sm = lax.fori_loop(0, _NT // 256, red1, accr)

    pbuf[0, :] = sy2
    pbuf[1, :] = syw
    pbuf[2, :] = sw2
    pbuf[3, :] = sm
    pbuf[4, :] = s_abs
    pltpu.sync_copy(pbuf, parts.at[c, s])


_mesh = plsc.VectorSubcoreMesh(core_axis_name="c", subcore_axis_name="s")

_sc_call = functools.partial(
    pl.kernel,
    out_type=(
        jax.ShapeDtypeStruct((_B, 16, 5, 16), jnp.float32),
        jax.ShapeDtypeStruct((_B, 16, _NP), jnp.float32),
        jax.ShapeDtypeStruct((_B, _NP), jnp.float32),
        jax.ShapeDtypeStruct((_B, _NP), jnp.float32),
    ),
    mesh=_mesh,
    scratch_types=[
        pltpu.VMEM((_NP,), jnp.float32),
        pltpu.VMEM((2, _GP), jnp.float32),
        pltpu.VMEM((2, _IP), jnp.int32),
        pltpu.VMEM((_NT,), jnp.float32),
        pltpu.VMEM((_NT,), jnp.float32),
        pltpu.VMEM((5, 16), jnp.float32),
        pltpu.SemaphoreType.DMA,
        pltpu.SemaphoreType.DMA,
        pltpu.SemaphoreType.DMA,
        pltpu.SemaphoreType.DMA,
    ],
    compiler_params=pltpu.CompilerParams(use_tc_tiling_on_sc=False,
                                         needs_layout_passes=False),
)(_sc_body)


def kernel(G_coeffs, A_diag, A_off, neighbors, valid_mask):
    b, n, _ = G_coeffs.shape
    pad = _NP - n
    wn = jax.random.normal(jax.random.key(42), (b, n), dtype=jnp.float32)
    m = valid_mask[:, :, 0]
    w = wn * m

    def panels(x, width):
        xp = jnp.pad(x, ((0, 0), (0, pad), (0, 0)))
        return xp.reshape(b, _NBLK, 128, width).swapaxes(2, 3).reshape(
            b, _NBLK, width * 128)

    gq = panels(G_coeffs, 25)
    aq = panels(jnp.concatenate([A_diag, A_off], axis=2), 25)
    nbq = panels(neighbors.astype(jnp.int32), 24)
    wp = jnp.pad(w, ((0, 0), (0, pad)))
    mp = jnp.pad(m, ((0, 0), (0, pad)))
    parts, _, _, _ = _sc_call(gq, aq, nbq, wp, mp)
    tot = parts.sum(axis=(0, 1, 3))
    sy2, syw, sw2, sm, sabs = tot[0], tot[1], tot[2], tot[3], tot[4]
    norm_a = sabs / (sm * 25 + 1e-6)
    d = norm_a + 1e-8
    loss = (sy2 / (d * d) - 2.0 * syw / d + sw2) / (sm + 1e-6)
    return loss
